# baseline (speedup 1.0000x reference)
import jax
import jax.numpy as jnp
from jax.experimental import pallas as pl
from jax.experimental.pallas import tpu as pltpu


def _copy_kernel(x_ref, o_ref, sem):
    pltpu.make_async_copy(x_ref, o_ref, sem).start()
    pltpu.make_async_copy(x_ref, o_ref, sem).wait()


def kernel(embeddings):
    return pl.pallas_call(
        _copy_kernel,
        out_shape=jax.ShapeDtypeStruct(embeddings.shape, embeddings.dtype),
        in_specs=[pl.BlockSpec(memory_space=pl.ANY)],
        out_specs=pl.BlockSpec(memory_space=pl.ANY),
        scratch_shapes=[pltpu.SemaphoreType.DMA],
    )(embeddings)


# speedup vs baseline: 3.2917x; 3.2917x over previous
import jax
import jax.numpy as jnp
from jax.experimental import pallas as pl
from jax.experimental.pallas import tpu as pltpu


def _copy_kernel(x_ref, o_ref, sem):
    pltpu.make_async_copy(x_ref, o_ref, sem).start()
    pltpu.make_async_copy(x_ref, o_ref, sem).wait()


def kernel(embeddings):
    n_rows, dim = embeddings.shape
    wide = embeddings.reshape(n_rows * dim // 128, 128)
    out = pl.pallas_call(
        _copy_kernel,
        out_shape=jax.ShapeDtypeStruct(wide.shape, wide.dtype),
        in_specs=[pl.BlockSpec(memory_space=pl.ANY)],
        out_specs=pl.BlockSpec(memory_space=pl.ANY),
        scratch_shapes=[pltpu.SemaphoreType.DMA],
    )(wide)
    return out.reshape(n_rows, dim)


# speedup vs baseline: 16.9063x; 5.1361x over previous
import jax
import jax.numpy as jnp
from jax import lax
from jax.experimental import pallas as pl
from jax.experimental.pallas import tpu as pltpu
from jax.experimental.pallas import tpu_sc as plsc

_NC = 2
_NS = 16
_NW = _NC * _NS

_ROWS = 1000000
_DIM = 32
_RPW = (_ROWS // _NW) // 8 * 8
_TAIL_BASE = _NW * _RPW
_TAIL_ROWS = _ROWS - _TAIL_BASE
_CHUNK = 504
_N_CHUNKS = _RPW // _CHUNK
_NBUF = 2


def _sc_copy(in_hbm, out_hbm, buf, load_sems, store_sems):
    wid = lax.axis_index("s") * _NC + lax.axis_index("c")
    base = pl.multiple_of(wid * _RPW, 8)

    store_copies = [None] * _N_CHUNKS
    for k in range(_N_CHUNKS):
        s = k % _NBUF
        if k >= _NBUF:
            store_copies[k - _NBUF].wait()
        src = in_hbm.at[pl.ds(base + k * _CHUNK, _CHUNK)]
        dst = out_hbm.at[pl.ds(base + k * _CHUNK, _CHUNK)]
        lc = pltpu.make_async_copy(src, buf.at[s], load_sems.at[s])
        lc.start()
        lc.wait()
        sc_ = pltpu.make_async_copy(buf.at[s], dst, store_sems.at[s])
        sc_.start()
        store_copies[k] = sc_
    for k in range(_N_CHUNKS - _NBUF, _N_CHUNKS):
        store_copies[k].wait()

    @pl.when(wid == 0)
    def _():
        tail = buf.at[0].at[pl.ds(0, _TAIL_ROWS)]
        pltpu.sync_copy(in_hbm.at[pl.ds(_TAIL_BASE, _TAIL_ROWS)], tail)
        pltpu.sync_copy(tail, out_hbm.at[pl.ds(_TAIL_BASE, _TAIL_ROWS)])


def kernel(embeddings):
    mesh = plsc.VectorSubcoreMesh(core_axis_name="c", subcore_axis_name="s")
    run = pl.kernel(
        _sc_copy,
        out_type=jax.ShapeDtypeStruct(embeddings.shape, embeddings.dtype),
        mesh=mesh,
        scratch_types=[
            pltpu.VMEM((_NBUF, _CHUNK, _DIM), jnp.float32),
            pltpu.SemaphoreType.DMA((_NBUF,)),
            pltpu.SemaphoreType.DMA((_NBUF,)),
        ],
    )
    return run(embeddings)
